# Initial kernel scaffold; baseline (speedup 1.0000x reference)
import jax
import jax.numpy as jnp
from jax.experimental import pallas as pl


def kernel(input_ids, table):
    raise NotImplementedError("write your pallas kernel here")


# speedup vs baseline: 2.4354x; 2.4354x over previous
import functools

import jax
import jax.numpy as jnp
from jax import lax
from jax.experimental import pallas as pl
from jax.experimental.pallas import tpu as pltpu
from jax.experimental.pallas import tpu_sc as plsc

_NUM_CORES = 2
_NUM_SUBCORES = 16
_NUM_WORKERS = _NUM_CORES * _NUM_SUBCORES
_CHUNK = 32


def _make_lookup(n_ids: int, vocab: int, d: int):
    assert n_ids % (_NUM_WORKERS * _CHUNK) == 0
    b_per_w = n_ids // _NUM_WORKERS
    n_chunks = b_per_w // _CHUNK
    mesh = plsc.VectorSubcoreMesh(core_axis_name="c", subcore_axis_name="s")

    @functools.partial(
        pl.kernel,
        out_type=jax.ShapeDtypeStruct((n_ids, d), jnp.float32),
        mesh=mesh,
        scratch_types=[
            pltpu.VMEM((b_per_w,), jnp.int32),
            pltpu.VMEM((_CHUNK, d), jnp.float32),
            pltpu.SemaphoreType.DMA,
        ],
    )
    def lookup(ids_hbm, table_hbm, out_hbm, idx_v, rows_v, sem):
        wid = lax.axis_index("s") * _NUM_CORES + lax.axis_index("c")
        base = wid * b_per_w
        pltpu.sync_copy(ids_hbm.at[pl.ds(base, b_per_w)], idx_v)

        def chunk_body(g, carry):
            off = g * _CHUNK
            pltpu.async_copy(
                table_hbm.at[idx_v.at[pl.ds(off, _CHUNK)]], rows_v, sem
            ).wait()
            pltpu.sync_copy(rows_v, out_hbm.at[pl.ds(base + off, _CHUNK)])
            return carry

        lax.fori_loop(0, n_chunks, chunk_body, 0)

    return lookup


def kernel(input_ids, table):
    b, s = input_ids.shape
    vocab, d = table.shape
    ids_flat = input_ids.reshape(-1)
    out = _make_lookup(b * s, vocab, d)(ids_flat, table)
    return out.reshape(b, s, d)


# speedup vs baseline: 2.8634x; 1.1758x over previous
import functools

import jax
import jax.numpy as jnp
from jax import lax
from jax.experimental import pallas as pl
from jax.experimental.pallas import tpu as pltpu
from jax.experimental.pallas import tpu_sc as plsc

_NUM_CORES = 2
_NUM_SUBCORES = 16
_NUM_WORKERS = _NUM_CORES * _NUM_SUBCORES
_CHUNK = 32


def _make_lookup(n_ids: int, vocab: int, d: int):
    assert n_ids % (_NUM_WORKERS * _CHUNK) == 0
    b_per_w = n_ids // _NUM_WORKERS
    n_chunks = b_per_w // _CHUNK
    mesh = plsc.VectorSubcoreMesh(core_axis_name="c", subcore_axis_name="s")

    @functools.partial(
        pl.kernel,
        out_type=jax.ShapeDtypeStruct((n_ids, d), jnp.float32),
        mesh=mesh,
        scratch_types=[
            pltpu.VMEM((b_per_w,), jnp.int32),
            pltpu.VMEM((_CHUNK, d), jnp.float32),
            pltpu.VMEM((_CHUNK, d), jnp.float32),
            pltpu.SemaphoreType.DMA,
            pltpu.SemaphoreType.DMA,
        ],
    )
    def lookup(ids_hbm, table_hbm, out_hbm, idx_v, buf0, buf1, sem0, sem1):
        wid = lax.axis_index("s") * _NUM_CORES + lax.axis_index("c")
        base = wid * b_per_w
        pltpu.sync_copy(ids_hbm.at[pl.ds(base, b_per_w)], idx_v)

        max_off = (n_chunks - 1) * _CHUNK

        def start_gather(g, buf, sem):
            off = lax.min(g * _CHUNK, max_off)
            pltpu.async_copy(
                table_hbm.at[idx_v.at[pl.ds(off, _CHUNK)]], buf, sem
            )

        def wait_gather(buf, sem):
            pltpu.make_async_copy(
                table_hbm.at[idx_v.at[pl.ds(0, _CHUNK)]], buf, sem
            ).wait()

        start_gather(0, buf0, sem0)

        def pair_body(h, carry):
            g0 = 2 * h
            start_gather(g0 + 1, buf1, sem1)
            wait_gather(buf0, sem0)
            pltpu.sync_copy(buf0, out_hbm.at[pl.ds(base + g0 * _CHUNK, _CHUNK)])
            start_gather(g0 + 2, buf0, sem0)
            wait_gather(buf1, sem1)
            pltpu.sync_copy(
                buf1, out_hbm.at[pl.ds(base + (g0 + 1) * _CHUNK, _CHUNK)]
            )
            return carry

        lax.fori_loop(0, n_chunks // 2, pair_body, 0)
        wait_gather(buf0, sem0)

    return lookup


def kernel(input_ids, table):
    b, s = input_ids.shape
    vocab, d = table.shape
    ids_flat = input_ids.reshape(-1)
    out = _make_lookup(b * s, vocab, d)(ids_flat, table)
    return out.reshape(b, s, d)


# speedup vs baseline: 2.8661x; 1.0010x over previous
import functools

import jax
import jax.numpy as jnp
from jax import lax
from jax.experimental import pallas as pl
from jax.experimental.pallas import tpu as pltpu
from jax.experimental.pallas import tpu_sc as plsc

_NUM_CORES = 2
_NUM_SUBCORES = 16
_NUM_WORKERS = _NUM_CORES * _NUM_SUBCORES
_CHUNK = 16
_NBUF = 4


def _make_lookup(n_ids: int, vocab: int, d: int):
    assert n_ids % (_NUM_WORKERS * _CHUNK) == 0
    b_per_w = n_ids // _NUM_WORKERS
    n_chunks = b_per_w // _CHUNK
    mesh = plsc.VectorSubcoreMesh(core_axis_name="c", subcore_axis_name="s")

    @functools.partial(
        pl.kernel,
        out_type=jax.ShapeDtypeStruct((n_ids, d), jnp.float32),
        mesh=mesh,
        scratch_types=[
            pltpu.VMEM((b_per_w,), jnp.int32),
            [pltpu.VMEM((_CHUNK, d), jnp.float32) for _ in range(_NBUF)],
            [pltpu.SemaphoreType.DMA for _ in range(_NBUF)],
            [pltpu.SemaphoreType.DMA for _ in range(_NBUF)],
        ],
    )
    def lookup(ids_hbm, table_hbm, out_hbm, idx_v, bufs, gsems, wsems):
        wid = lax.axis_index("s") * _NUM_CORES + lax.axis_index("c")
        base = wid * b_per_w
        pltpu.sync_copy(ids_hbm.at[pl.ds(base, b_per_w)], idx_v)

        max_off = (n_chunks - 1) * _CHUNK

        def start_gather(g, b):
            off = lax.min(g * _CHUNK, max_off)
            pltpu.async_copy(
                table_hbm.at[idx_v.at[pl.ds(off, _CHUNK)]], bufs[b], gsems[b]
            )

        def wait_gather(b):
            pltpu.make_async_copy(
                table_hbm.at[idx_v.at[pl.ds(0, _CHUNK)]], bufs[b], gsems[b]
            ).wait()

        def start_write(g, b):
            pltpu.async_copy(
                bufs[b], out_hbm.at[pl.ds(base + g * _CHUNK, _CHUNK)], wsems[b]
            )

        def wait_write(b):
            pltpu.make_async_copy(
                bufs[b], out_hbm.at[pl.ds(base, _CHUNK)], wsems[b]
            ).wait()

        start_gather(0, 0)
        start_gather(1, 1)

        def step(g, b, first_round):
            bp = (b + 2) % _NBUF
            if not first_round:
                wait_write(bp)
            start_gather(g + 2, bp)
            wait_gather(b)
            start_write(g, b)

        step(0, 0, True)
        step(1, 1, True)
        step(2, 2, False)
        step(3, 3, False)

        def quad_body(h, carry):
            g0 = 4 * h
            for b in range(_NBUF):
                step(g0 + b, b, False)
            return carry

        lax.fori_loop(1, n_chunks // _NBUF, quad_body, 0)

        wait_write(2)
        wait_write(3)
        wait_gather(0)
        wait_gather(1)

    return lookup


def kernel(input_ids, table):
    b, s = input_ids.shape
    vocab, d = table.shape
    ids_flat = input_ids.reshape(-1)
    out = _make_lookup(b * s, vocab, d)(ids_flat, table)
    return out.reshape(b, s, d)
